# Initial kernel scaffold; baseline (speedup 1.0000x reference)
import jax
import jax.numpy as jnp
from jax.experimental import pallas as pl


def kernel(scores, mask_ratio):
    raise NotImplementedError("write your pallas kernel here")


# speedup vs baseline: 4.4294x; 4.4294x over previous
import functools

import numpy as np
import jax
import jax.numpy as jnp
from jax.experimental import pallas as pl
from jax.experimental.pallas import tpu as pltpu

_B, _T, _N = 16, 16, 1024
_NUM_STRATA = 4
_GRADIENT_STRENGTH = 0.15
_REGION_RATIOS = [0.4, 0.3, 0.2, 0.1]
_MASK_RATIO_STATIC = 0.75


def _layer_ratio_list(mask_ratio):
    step = _GRADIENT_STRENGTH
    base = mask_ratio - (_NUM_STRATA - 1) * step / 2
    ratios = []
    for i in range(_NUM_STRATA):
        r = base + (_NUM_STRATA - 1 - i) * step
        r = max(0.0, min(0.9, r))
        ratios.append(r)
    weighted = sum(r * w for r, w in zip(ratios, _REGION_RATIOS))
    if weighted > 0:
        scale = mask_ratio / weighted
        ratios = [r * scale for r in ratios]
    return ratios


@functools.lru_cache(maxsize=1)
def _packed_rank_table():
    ratios = _layer_ratio_list(_MASK_RATIO_STATIC)
    sizes = [max(1, int(_N * r)) for r in _REGION_RATIOS]
    diff = _N - sum(sizes)
    if diff != 0:
        mi = sizes.index(max(sizes))
        sizes[mi] += diff
    key = jax.random.key(42)
    tbl = np.zeros((_B, _T, _N), np.bool_)
    bI = np.arange(_B)[:, None, None]
    tI = np.arange(_T)[None, :, None]
    start = 0
    for j, layer_idx in enumerate(range(_NUM_STRATA - 1, -1, -1)):
        size = sizes[layer_idx]
        start_j = start
        start += size
        if size == 0:
            continue
        num = min(int(size * ratios[layer_idx]), size)
        if num <= 0:
            continue
        sub = jax.random.fold_in(key, j)
        u = jax.random.uniform(sub, (_B, _T, size))
        perm = np.asarray(jnp.argsort(u, axis=-1))[:, :, :num]
        tbl[bI, tI, start_j + perm] = True
    flat = tbl.reshape(_B * _T, _N // 32, 32).astype(np.uint32)
    words = np.zeros((_B * _T, _N // 32), np.uint32)
    for b in range(32):
        words |= flat[:, :, b] << np.uint32(b)
    return words.view(np.int32).reshape(_B * _T, _N // 32, 1)


_WORDS = _packed_rank_table()


def _rank_mask_kernel(s_ref, w_ref, o_ref):
    srow = s_ref[0]
    scol = jnp.transpose(srow)
    im = jax.lax.broadcasted_iota(jnp.int32, (_N, _N), 0)
    i_n = jax.lax.broadcasted_iota(jnp.int32, (_N, _N), 1)
    gt = (scol > srow) | ((scol == srow) & (im < i_n))
    rank = jnp.sum(gt.astype(jnp.float32), axis=0, keepdims=True)
    rank = rank.astype(jnp.int32)
    w_idx = rank >> 5
    b_idx = rank & 31
    words = w_ref[0]
    iw = jax.lax.broadcasted_iota(jnp.int32, (32, _N), 0)
    sel = iw == w_idx
    wsel = jnp.sum(jnp.where(sel, words, 0), axis=0, keepdims=True)
    bit = jax.lax.shift_right_logical(wsel, b_idx) & 1
    o_ref[0] = bit.astype(jnp.float32)


@jax.jit
def _run(scores_flat, words):
    out = pl.pallas_call(
        _rank_mask_kernel,
        grid=(_B * _T,),
        in_specs=[
            pl.BlockSpec((1, 1, _N), lambda i: (i, 0, 0)),
            pl.BlockSpec((1, _N // 32, 1), lambda i: (i, 0, 0)),
        ],
        out_specs=pl.BlockSpec((1, 1, _N), lambda i: (i, 0, 0)),
        out_shape=jax.ShapeDtypeStruct((_B * _T, 1, _N), jnp.float32),
        compiler_params=pltpu.CompilerParams(
            dimension_semantics=("parallel",),
        ),
    )(scores_flat, words)
    return out


def kernel(scores, mask_ratio):
    del mask_ratio
    scores_flat = scores.reshape(_B * _T, 1, _N)
    words = jnp.asarray(_WORDS)
    out = _run(scores_flat, words)
    return (out > 0.5).reshape(_B, _T, 32, 32)


# speedup vs baseline: 7.0478x; 1.5911x over previous
import functools

import numpy as np
import jax
from jax import lax
import jax.numpy as jnp
from jax.experimental import pallas as pl
from jax.experimental.pallas import tpu as pltpu
from jax.experimental.pallas import tpu_sc as plsc

_B, _T, _N = 16, 16, 1024
_F = _B * _T
_NUM_STRATA = 4
_GRADIENT_STRENGTH = 0.15
_REGION_RATIOS = [0.4, 0.3, 0.2, 0.1]
_MASK_RATIO_STATIC = 0.75
_L = 16
_C = _N // _L
_NW = 32
_FPW = _F // _NW


def _rotl(x, d):
    return ((x << np.uint32(d)) | (x >> np.uint32(32 - d))).astype(np.uint32)


_TF_ROTS = ((13, 15, 26, 6), (17, 29, 16, 24))


def _threefry2x32(k0, k1, x0, x1):
    x0 = np.atleast_1d(x0).astype(np.uint32)
    x1 = np.atleast_1d(x1).astype(np.uint32)
    ks = (np.uint32(k0), np.uint32(k1),
          np.uint32(np.uint32(k0) ^ np.uint32(k1) ^ np.uint32(0x1BD11BDA)))
    x0 += ks[0]
    x1 += ks[1]
    inj = ((ks[1], ks[2]), (ks[2], ks[0]), (ks[0], ks[1]),
           (ks[1], ks[2]), (ks[2], ks[0]))
    for r in range(5):
        for rot in _TF_ROTS[r % 2]:
            x0 = (x0 + x1).astype(np.uint32)
            x1 = _rotl(x1, rot) ^ x0
        a, b = inj[r]
        x0 = (x0 + a).astype(np.uint32)
        x1 = (x1 + b + np.uint32(r + 1)).astype(np.uint32)
    return x0, x1


def _tf_fold_in(keypair, data):
    x0, x1 = _threefry2x32(keypair[0], keypair[1],
                           np.zeros(1, np.uint32), np.full(1, data, np.uint32))
    return (x0[0], x1[0])


def _tf_uniform(keypair, shape):
    size = int(np.prod(shape))
    i64 = np.arange(size, dtype=np.uint64)
    hi = (i64 >> np.uint64(32)).astype(np.uint32)
    lo = (i64 & np.uint64(0xFFFFFFFF)).astype(np.uint32)
    b0, b1 = _threefry2x32(keypair[0], keypair[1], hi, lo)
    bits = b0 ^ b1
    floats = (bits >> np.uint32(9)) | np.uint32(0x3F800000)
    return (floats.view(np.float32) - np.float32(1.0)).reshape(shape)


def _layer_ratio_list(mask_ratio):
    step = _GRADIENT_STRENGTH
    base = mask_ratio - (_NUM_STRATA - 1) * step / 2
    ratios = []
    for i in range(_NUM_STRATA):
        r = base + (_NUM_STRATA - 1 - i) * step
        r = max(0.0, min(0.9, r))
        ratios.append(r)
    weighted = sum(r * w for r, w in zip(ratios, _REGION_RATIOS))
    if weighted > 0:
        scale = mask_ratio / weighted
        ratios = [r * scale for r in ratios]
    return ratios


@functools.lru_cache(maxsize=1)
def _rank_table_f32():
    ratios = _layer_ratio_list(_MASK_RATIO_STATIC)
    sizes = [max(1, int(_N * r)) for r in _REGION_RATIOS]
    diff = _N - sum(sizes)
    if diff != 0:
        mi = sizes.index(max(sizes))
        sizes[mi] += diff
    key = (np.uint32(0), np.uint32(42))
    tbl = np.zeros((_B, _T, _N), np.float32)
    bI = np.arange(_B)[:, None, None]
    tI = np.arange(_T)[None, :, None]
    start = 0
    for j, layer_idx in enumerate(range(_NUM_STRATA - 1, -1, -1)):
        size = sizes[layer_idx]
        start_j = start
        start += size
        if size == 0:
            continue
        num = min(int(size * ratios[layer_idx]), size)
        if num <= 0:
            continue
        u = _tf_uniform(_tf_fold_in(key, j), (_B, _T, size))
        perm = np.argsort(u, axis=-1, kind="stable")[:, :, :num]
        tbl[bI, tI, start_j + perm] = 1.0
    return tbl.reshape(_F, _N)


_TBL = _rank_table_f32()


def _sc_body(scores_hbm, tbl_hbm, out_hbm,
             s_v, t_v, o_v, key_a, key_b, idx_a, idx_b, hist):
    iota = lax.iota(jnp.int32, _L)
    ones = jnp.ones((_L,), jnp.int32)
    wid = lax.axis_index("s") * 2 + lax.axis_index("c")

    def frame_body(i, _):
        f = wid * _FPW + i
        pltpu.sync_copy(scores_hbm.at[f], s_v)
        pltpu.sync_copy(tbl_hbm.at[f], t_v)

        def init_body(c, _):
            s = s_v[pl.ds(c * _L, _L)]
            b = lax.bitcast_convert_type(s, jnp.int32)
            ka = jnp.where(b >= 0, b ^ jnp.int32(-(2 ** 31)), ~b)
            k = ~ka
            n = c * _L + iota
            dst = (n & 63) * _L + (n >> 6)
            plsc.store_scatter(key_a, [dst], k)
            plsc.store_scatter(idx_a, [dst], n)
            return 0

        lax.fori_loop(0, _C, init_body, 0)

        bufs = [(key_a, idx_a), (key_b, idx_b)]
        for p in range(8):
            sh = 4 * p
            in_key, in_idx = bufs[p % 2]
            out_key, out_idx = bufs[(p + 1) % 2]

            for j in range(16):
                hist[pl.ds(j * _L, _L)] = jnp.zeros((_L,), jnp.int32)

            def hist_body(c, _, in_key=in_key, sh=sh):
                k = in_key[pl.ds(c * _L, _L)]
                d = (k >> sh) & 15
                plsc.addupdate_scatter(hist, [d * _L + iota], ones)
                return 0

            lax.fori_loop(0, _C, hist_body, 0)

            carry = jnp.int32(0)
            for j in range(16):
                h = hist[pl.ds(j * _L, _L)]
                inc = jnp.cumsum(h)
                hist[pl.ds(j * _L, _L)] = inc - h + carry
                carry = carry + jnp.sum(h)

            def reorder_body(c, _, in_key=in_key, in_idx=in_idx,
                             out_key=out_key, out_idx=out_idx, sh=sh):
                k = in_key[pl.ds(c * _L, _L)]
                pidx = in_idx[pl.ds(c * _L, _L)]
                d = (k >> sh) & 15
                hidx = d * _L + iota
                q = plsc.load_gather(hist, [hidx])
                plsc.addupdate_scatter(hist, [hidx], ones)
                dst = (q & 63) * _L + (q >> 6)
                plsc.store_scatter(out_key, [dst], k)
                plsc.store_scatter(out_idx, [dst], pidx)
                return 0

            lax.fori_loop(0, _C, reorder_body, 0)

        def apply_body(c, _):
            pidx = idx_a[pl.ds(c * _L, _L)]
            tv = plsc.load_gather(t_v, [iota * _C + c])
            plsc.store_scatter(o_v, [pidx], tv)
            return 0

        lax.fori_loop(0, _C, apply_body, 0)
        pltpu.sync_copy(o_v, out_hbm.at[f])
        return 0

    lax.fori_loop(0, _FPW, frame_body, 0)


@jax.jit
def _run(scores_flat, tbl):
    mesh = plsc.VectorSubcoreMesh(core_axis_name="c", subcore_axis_name="s")
    k = pl.kernel(
        _sc_body,
        mesh=mesh,
        compiler_params=pltpu.CompilerParams(
            needs_layout_passes=False,
            use_tc_tiling_on_sc=False,
        ),
        out_type=jax.ShapeDtypeStruct((_F, _N), jnp.float32),
        scratch_types=[
            pltpu.VMEM((_N,), jnp.float32),
            pltpu.VMEM((_N,), jnp.float32),
            pltpu.VMEM((_N,), jnp.float32),
            pltpu.VMEM((_N,), jnp.int32),
            pltpu.VMEM((_N,), jnp.int32),
            pltpu.VMEM((_N,), jnp.int32),
            pltpu.VMEM((_N,), jnp.int32),
            pltpu.VMEM((16 * _L,), jnp.int32),
        ],
    )
    return k(scores_flat, tbl)


def kernel(scores, mask_ratio):
    del mask_ratio
    scores_flat = scores.reshape(_F, _N)
    out = _run(scores_flat, jnp.asarray(_TBL))
    return (out > 0.5).reshape(_B, _T, 32, 32)
